# baseline (speedup 1.0000x reference)
import functools
import jax
import jax.numpy as jnp
import numpy as np
from jax import lax
from jax.experimental import pallas as pl
from jax.experimental.pallas import tpu as pltpu
from jax.experimental.pallas import tpu_sc as plsc

B = 2048
D = 2048
E = 8
NC = 16
FB = 16
FPE = 2
H = 1024
RH = 1024
K = 2
FD = NC * FB
EF = FPE * FB
RIN = D + FD

M = 256
NT = (B * K) // M + E - 1 + 1
NP = NT * M

NW = 32
TPW = B // NW
CH = 16

INTERPRET = False

_FSEL = np.zeros((E, 128, EF), np.float32)
for _e in range(E):
    for _j in range(EF):
        _FSEL[_e, 32 * (_e % 4) + _j, _j] = 1.0


def _gelu(x):
    return x * 0.5 * (1.0 + jax.lax.erf(x * np.float32(0.7071067811865476)))


def _router_body(hid_ref, ft_ref, gam_ref, bet_ref, w1_ref, b1_ref, w2_ref,
                 b2_ref, h_ref, g1_ref, g2_ref, i1_ref, i2_ref, r1o_ref,
                 r2o_ref, cnt_ref, te_ref, carry_ref):
    x = hid_ref[...]
    mu = jnp.mean(x, axis=-1, keepdims=True)
    var = jnp.mean((x - mu) ** 2, axis=-1, keepdims=True)
    h = (x - mu) * jax.lax.rsqrt(var + 1e-5) * gam_ref[...] + bet_ref[...]
    h_ref[...] = h
    r1 = jnp.dot(h, w1_ref[:D], preferred_element_type=jnp.float32)
    r1 = r1 + jnp.dot(ft_ref[...], w1_ref[D:], preferred_element_type=jnp.float32)
    r1 = _gelu(r1 + b1_ref[...])
    logits = jnp.dot(r1, w2_ref[...], preferred_element_type=jnp.float32)
    logits = logits + b2_ref[...]
    ii = jax.lax.broadcasted_iota(jnp.int32, logits.shape, 1)
    v1 = jnp.max(logits, axis=-1, keepdims=True)
    i1 = jnp.min(jnp.where(logits == v1, ii, E), axis=-1, keepdims=True)
    ml = jnp.where(ii == i1, -jnp.inf, logits)
    v2 = jnp.max(ml, axis=-1, keepdims=True)
    i2 = jnp.min(jnp.where(ml == v2, ii, E), axis=-1, keepdims=True)
    e2 = jnp.exp(v2 - v1)
    inv = 1.0 / (1.0 + e2)
    g1_ref[...] = jnp.broadcast_to(inv, (inv.shape[0], 128))
    g2_ref[...] = jnp.broadcast_to(e2 * inv, (inv.shape[0], 128))
    i1_ref[...] = i1
    i2_ref[...] = i2

    @pl.when(pl.program_id(0) == 0)
    def _():
        carry_ref[...] = jnp.zeros_like(carry_ref)

    bm = i1.shape[0]
    ii16 = jax.lax.broadcasted_iota(jnp.int32, (bm, 16), 1)
    oh1 = (ii16 == i1).astype(jnp.float32)
    oh2 = (ii16 == i2).astype(jnp.float32)
    rr = jax.lax.broadcasted_iota(jnp.int32, (bm, bm), 0)
    cc = jax.lax.broadcasted_iota(jnp.int32, (bm, bm), 1)
    tri = (cc <= rr).astype(jnp.float32)
    csum1 = jnp.dot(tri, oh1, preferred_element_type=jnp.float32)
    csum2 = jnp.dot(tri, oh2, preferred_element_type=jnp.float32)
    total1 = jnp.sum(oh1, axis=0, keepdims=True)
    total2 = jnp.sum(oh2, axis=0, keepdims=True)
    carry = carry_ref[...]
    rank1 = jnp.sum(jnp.where(oh1 > 0, csum1 + carry - 1.0, 0.0),
                    axis=1, keepdims=True)
    rank2 = jnp.sum(jnp.where(oh2 > 0, csum2 + carry + total1 - 1.0, 0.0),
                    axis=1, keepdims=True)
    r1o_ref[...] = rank1.astype(jnp.int32)
    r2o_ref[...] = rank2.astype(jnp.int32)
    new_carry = carry + total1 + total2
    carry_ref[...] = new_carry

    tilesf = jnp.floor((new_carry + (M - 1.0)) * (1.0 / M))
    r16 = jax.lax.broadcasted_iota(jnp.int32, (16, 16), 0)
    c16 = jax.lax.broadcasted_iota(jnp.int32, (16, 16), 1)
    triu = (r16 <= c16).astype(jnp.float32)
    incl = jnp.dot(tilesf, triu, preferred_element_type=jnp.float32)
    starts = incl - tilesf
    cnt_ref[...] = starts.astype(jnp.int32)

    lane16 = jax.lax.broadcasted_iota(jnp.int32, (1, 16), 1)
    tt = jax.lax.broadcasted_iota(jnp.int32, (32, 16), 0).astype(jnp.float32)
    cmp = jnp.where((starts <= tt) & (lane16 < E), 1.0, 0.0)
    te = jnp.sum(cmp, axis=1, keepdims=True) - 1.0
    n_tiles = jnp.sum(jnp.where(lane16 == E, starts, 0.0), axis=1,
                      keepdims=True)
    row32 = jax.lax.broadcasted_iota(jnp.int32, (32, 1), 0)
    te_ref[...] = jnp.where(row32 == 31, n_tiles, te).astype(jnp.int32)


def _run_router(hidden, feats, ln_gamma, ln_beta, rW1, rb1, rW2, rb2):
    bm = 256
    grid = (B // bm,)
    out_shapes = (
        jax.ShapeDtypeStruct((B, D), jnp.float32),
        jax.ShapeDtypeStruct((B, 128), jnp.float32),
        jax.ShapeDtypeStruct((B, 128), jnp.float32),
        jax.ShapeDtypeStruct((B, 1), jnp.int32),
        jax.ShapeDtypeStruct((B, 1), jnp.int32),
        jax.ShapeDtypeStruct((B, 1), jnp.int32),
        jax.ShapeDtypeStruct((B, 1), jnp.int32),
        jax.ShapeDtypeStruct((1, 16), jnp.int32),
        jax.ShapeDtypeStruct((32, 1), jnp.int32),
    )

    def bspec():
        return pl.BlockSpec((bm, 1), lambda i: (i, 0))

    return pl.pallas_call(
        _router_body,
        grid=grid,
        in_specs=[
            pl.BlockSpec((bm, D), lambda i: (i, 0)),
            pl.BlockSpec((bm, FD), lambda i: (i, 0)),
            pl.BlockSpec((D,), lambda i: (0,)),
            pl.BlockSpec((D,), lambda i: (0,)),
            pl.BlockSpec((RIN, RH), lambda i: (0, 0)),
            pl.BlockSpec((RH,), lambda i: (0,)),
            pl.BlockSpec((RH, E), lambda i: (0, 0)),
            pl.BlockSpec((E,), lambda i: (0,)),
        ],
        out_specs=(
            pl.BlockSpec((bm, D), lambda i: (i, 0)),
            pl.BlockSpec((bm, 128), lambda i: (i, 0)),
            pl.BlockSpec((bm, 128), lambda i: (i, 0)),
            bspec(), bspec(), bspec(), bspec(),
            pl.BlockSpec((1, 16), lambda i: (0, 0)),
            pl.BlockSpec((32, 1), lambda i: (0, 0)),
        ),
        out_shape=out_shapes,
        scratch_shapes=[pltpu.VMEM((1, 16), jnp.float32)],
        interpret=INTERPRET,
    )(hidden, feats, ln_gamma, ln_beta, rW1, rb1, rW2, rb2)


def _dispatch_body(cnt_hbm, i1_hbm, i2_hbm, r1_hbm, r2_hbm, g1_hbm, g2_hbm,
                   h_hbm, f2_hbm,
                   xh_hbm, xf_hbm, s1_hbm, s2_hbm,
                   rsg_v, ids1_v, rnk1_v, ids2_v, rnk2_v, sl1_v, sl2_v,
                   rows_v, fr_v, sem_h, sem_f, sem_fs, sem_x, sem_g):
    wid = lax.axis_index("s") * 2 + lax.axis_index("c")
    base = wid * TPW
    for ik_hbm, rk_hbm, sk_hbm, ids_v, rnk_v, sl_v in (
            (i1_hbm, r1_hbm, s1_hbm, ids1_v, rnk1_v, sl1_v),
            (i2_hbm, r2_hbm, s2_hbm, ids2_v, rnk2_v, sl2_v)):
        pltpu.sync_copy(ik_hbm.at[pl.ds(base, TPW)], ids_v)
        pltpu.sync_copy(rk_hbm.at[pl.ds(base, TPW)], rnk_v)
        for q in range(TPW // CH):
            sl = pl.ds(q * CH, CH)
            pltpu.sync_copy(cnt_hbm.at[ids_v[sl]], rsg_v)
            sl_v[sl] = (rsg_v[...] << 8) + rnk_v[sl]
        pltpu.sync_copy(sl_v, sk_hbm.at[pl.ds(base, TPW)])

    for j in range(TPW // CH):
        off = base + j * CH
        sl = pl.ds(j * CH, CH)
        ch = pltpu.async_copy(h_hbm.at[pl.ds(off, CH)], rows_v, sem_h)
        for k in range(2):
            gk_hbm = g1_hbm if k == 0 else g2_hbm
            ids_v = ids1_v if k == 0 else ids2_v
            sl_v = sl1_v if k == 0 else sl2_v
            slot = sl_v[sl]
            cgl = pltpu.async_copy(gk_hbm.at[pl.ds(off, CH)],
                                   fr_v.at[:, pl.ds(128, 128)], sem_g)
            toks = jax.lax.broadcasted_iota(jnp.int32, (16,), 0) + off
            fidx = toks * 2 + (ids_v[sl] >> 2)
            cf = pltpu.async_copy(f2_hbm.at[fidx],
                                  fr_v.at[:, pl.ds(0, 128)], sem_f)
            cgl.wait()
            cf.wait()
            cf2 = pltpu.async_copy(fr_v, xf_hbm.at[slot], sem_fs)
            if k == 0:
                ch.wait()
            cx = pltpu.async_copy(rows_v, xh_hbm.at[slot], sem_x)
            cf2.wait()
            cx.wait()


def _run_dispatch(cnt16, i1, i2, r1, r2, g1b, g2b, h, f2):
    mesh = plsc.VectorSubcoreMesh(core_axis_name="c", subcore_axis_name="s")
    out_type = (
        jax.ShapeDtypeStruct((NP, D), jnp.float32),
        jax.ShapeDtypeStruct((NP, 256), jnp.float32),
        jax.ShapeDtypeStruct((B,), jnp.int32),
        jax.ShapeDtypeStruct((B,), jnp.int32),
    )
    f = pl.kernel(
        _dispatch_body,
        mesh=mesh,
        out_type=out_type,
        scratch_types=[
            pltpu.VMEM((16,), jnp.int32),
            pltpu.VMEM((TPW,), jnp.int32),
            pltpu.VMEM((TPW,), jnp.int32),
            pltpu.VMEM((TPW,), jnp.int32),
            pltpu.VMEM((TPW,), jnp.int32),
            pltpu.VMEM((TPW,), jnp.int32),
            pltpu.VMEM((TPW,), jnp.int32),
            pltpu.VMEM((CH, D), jnp.float32),
            pltpu.VMEM((CH, 256), jnp.float32),
            pltpu.SemaphoreType.DMA,
            pltpu.SemaphoreType.DMA,
            pltpu.SemaphoreType.DMA,
            pltpu.SemaphoreType.DMA,
            pltpu.SemaphoreType.DMA,
        ],
    )
    return f(cnt16, i1, i2, r1, r2, g1b, g2b, h, f2)


def _bdot(a, b):
    return jnp.dot(a.astype(jnp.bfloat16), b.astype(jnp.bfloat16),
                   preferred_element_type=jnp.float32)


def _expert_body(te_ref, xh_ref, xf_ref, w1h_ref, w1f_ref, b1_ref,
                 w2_ref, b2_ref, w3_ref, b3_ref, y_ref):
    @pl.when(pl.program_id(0) < te_ref[31])
    def _():
        x1 = _bdot(xh_ref[...], w1h_ref[0])
        x1 = x1 + _bdot(xf_ref[:, :128], w1f_ref[0])
        h1 = _gelu(x1 + b1_ref[0])
        h2 = _gelu(_bdot(h1, w2_ref[0]) + b2_ref[0])
        y = _bdot(h2, w3_ref[0]) + b3_ref[0]
        y_ref[...] = y * xf_ref[:, 128:129]


def _run_experts(te, xh, xf, We1, W1fp, be1, We2, be2, We3, be3):
    grid_spec = pltpu.PrefetchScalarGridSpec(
        num_scalar_prefetch=1,
        grid=(NT,),
        in_specs=[
            pl.BlockSpec((M, D), lambda i, te: (i, 0)),
            pl.BlockSpec((M, 256), lambda i, te: (i, 0)),
            pl.BlockSpec((1, D, H), lambda i, te: (te[i], 0, 0)),
            pl.BlockSpec((1, 128, H), lambda i, te: (te[i], 0, 0)),
            pl.BlockSpec((1, 1, H), lambda i, te: (te[i], 0, 0)),
            pl.BlockSpec((1, H, H), lambda i, te: (te[i], 0, 0)),
            pl.BlockSpec((1, 1, H), lambda i, te: (te[i], 0, 0)),
            pl.BlockSpec((1, H, D), lambda i, te: (te[i], 0, 0)),
            pl.BlockSpec((1, 1, D), lambda i, te: (te[i], 0, 0)),
        ],
        out_specs=pl.BlockSpec((M, D), lambda i, te: (i, 0)),
    )
    return pl.pallas_call(
        _expert_body,
        grid_spec=grid_spec,
        out_shape=jax.ShapeDtypeStruct((NP, D), jnp.float32),
        interpret=INTERPRET,
    )(te, xh, xf, We1, W1fp, be1, We2, be2, We3, be3)


def _combine_body(hid_hbm, yg_hbm, s1_hbm, s2_hbm, al_hbm,
                  out_hbm, s1_v, s2_v, y1_v, y2_v, hid_v, al_v, sem, sem2,
                  sem3, semw):
    wid = lax.axis_index("s") * 2 + lax.axis_index("c")
    base = wid * TPW
    pltpu.sync_copy(al_hbm, al_v)
    av = al_v[...]
    pltpu.sync_copy(s1_hbm.at[pl.ds(base, TPW)], s1_v)
    pltpu.sync_copy(s2_hbm.at[pl.ds(base, TPW)], s2_v)
    cw = None
    for j in range(TPW // CH):
        off = base + j * CH
        sl = pl.ds(j * CH, CH)
        c1 = pltpu.async_copy(yg_hbm.at[s1_v[sl]], y1_v, sem)
        c2 = pltpu.async_copy(yg_hbm.at[s2_v[sl]], y2_v, sem2)
        if cw is not None:
            cw.wait()
        c3 = pltpu.async_copy(hid_hbm.at[pl.ds(off, CH)], hid_v, sem3)
        c1.wait()
        c2.wait()
        c3.wait()
        for r in range(CH):
            def body(c, _):
                for u in range(8):
                    s = pl.ds(c * 128 + u * 16, 16)
                    hid_v[r, s] = hid_v[r, s] + av * (y1_v[r, s] + y2_v[r, s])
                return 0
            lax.fori_loop(0, D // 128, body, 0)
        cw = pltpu.async_copy(hid_v, out_hbm.at[pl.ds(off, CH)], semw)
    cw.wait()


def _run_combine(hidden, yg, s1, s2, alpha16):
    mesh = plsc.VectorSubcoreMesh(core_axis_name="c", subcore_axis_name="s")
    f = pl.kernel(
        _combine_body,
        mesh=mesh,
        out_type=jax.ShapeDtypeStruct((B, D), jnp.float32),
        scratch_types=[
            pltpu.VMEM((TPW,), jnp.int32),
            pltpu.VMEM((TPW,), jnp.int32),
            pltpu.VMEM((CH, D), jnp.float32),
            pltpu.VMEM((CH, D), jnp.float32),
            pltpu.VMEM((CH, D), jnp.float32),
            pltpu.VMEM((16,), jnp.float32),
            pltpu.SemaphoreType.DMA,
            pltpu.SemaphoreType.DMA,
            pltpu.SemaphoreType.DMA,
            pltpu.SemaphoreType.DMA,
        ],
    )
    return f(hidden, yg, s1, s2, alpha16)


def kernel(hidden, feature_bank, expert_bank_idx, ln_gamma, ln_beta,
           rW1, rb1, rW2, rb2, We1, be1, We2, be2, We3, be3, alpha):
    feats = feature_bank.reshape(B, FD)
    h, g1, g2, i1, i2, r1, r2, cnt, te = _run_router(
        hidden, feats, ln_gamma, ln_beta, rW1, rb1, rW2, rb2)

    f2 = feats.reshape(B * 2, 128)
    xh, xf, s1, s2 = _run_dispatch(
        cnt.reshape(16), i1.reshape(B), i2.reshape(B), r1.reshape(B),
        r2.reshape(B), g1, g2, h, f2)

    W1fp = jnp.matmul(_FSEL, We1[:, D:, :])
    yg = _run_experts(te.reshape(32), xh, xf, We1, W1fp,
                      be1.reshape(E, 1, H), We2, be2.reshape(E, 1, H),
                      We3, be3.reshape(E, 1, D))

    alpha16 = jnp.full((16,), 1.0, jnp.float32) * alpha
    return _run_combine(hidden, yg, s1, s2, alpha16)


# speedup vs baseline: 1.0092x; 1.0092x over previous
import functools
import jax
import jax.numpy as jnp
import numpy as np
from jax import lax
from jax.experimental import pallas as pl
from jax.experimental.pallas import tpu as pltpu
from jax.experimental.pallas import tpu_sc as plsc

B = 2048
D = 2048
E = 8
NC = 16
FB = 16
FPE = 2
H = 1024
RH = 1024
K = 2
FD = NC * FB
EF = FPE * FB
RIN = D + FD

M = 256
NT = (B * K) // M + E - 1 + 1
NP = NT * M

NW = 32
TPW = B // NW
CH = 16

INTERPRET = False

_FSEL = np.zeros((E, 128, EF), np.float32)
for _e in range(E):
    for _j in range(EF):
        _FSEL[_e, 32 * (_e % 4) + _j, _j] = 1.0


def _gelu(x):
    return x * 0.5 * (1.0 + jax.lax.erf(x * np.float32(0.7071067811865476)))


def _router_body(hid_ref, ft_ref, gam_ref, bet_ref, w1_ref, b1_ref, w2_ref,
                 b2_ref, h_ref, g1_ref, g2_ref, i1_ref, i2_ref, r1o_ref,
                 r2o_ref, cnt_ref, te_ref, carry_ref):
    x = hid_ref[...]
    mu = jnp.mean(x, axis=-1, keepdims=True)
    var = jnp.mean((x - mu) ** 2, axis=-1, keepdims=True)
    h = (x - mu) * jax.lax.rsqrt(var + 1e-5) * gam_ref[...] + bet_ref[...]
    h_ref[...] = h
    r1 = jnp.dot(h, w1_ref[:D], preferred_element_type=jnp.float32)
    r1 = r1 + jnp.dot(ft_ref[...], w1_ref[D:], preferred_element_type=jnp.float32)
    r1 = _gelu(r1 + b1_ref[...])
    logits = jnp.dot(r1, w2_ref[...], preferred_element_type=jnp.float32)
    logits = logits + b2_ref[...]
    ii = jax.lax.broadcasted_iota(jnp.int32, logits.shape, 1)
    v1 = jnp.max(logits, axis=-1, keepdims=True)
    i1 = jnp.min(jnp.where(logits == v1, ii, E), axis=-1, keepdims=True)
    ml = jnp.where(ii == i1, -jnp.inf, logits)
    v2 = jnp.max(ml, axis=-1, keepdims=True)
    i2 = jnp.min(jnp.where(ml == v2, ii, E), axis=-1, keepdims=True)
    e2 = jnp.exp(v2 - v1)
    inv = 1.0 / (1.0 + e2)
    g1_ref[...] = jnp.broadcast_to(inv, (inv.shape[0], 128))
    g2_ref[...] = jnp.broadcast_to(e2 * inv, (inv.shape[0], 128))
    i1_ref[...] = i1
    i2_ref[...] = i2

    @pl.when(pl.program_id(0) == 0)
    def _():
        carry_ref[...] = jnp.zeros_like(carry_ref)

    bm = i1.shape[0]
    ii16 = jax.lax.broadcasted_iota(jnp.int32, (bm, 16), 1)
    oh1 = (ii16 == i1).astype(jnp.float32)
    oh2 = (ii16 == i2).astype(jnp.float32)
    rr = jax.lax.broadcasted_iota(jnp.int32, (bm, bm), 0)
    cc = jax.lax.broadcasted_iota(jnp.int32, (bm, bm), 1)
    tri = (cc <= rr).astype(jnp.float32)
    csum1 = jnp.dot(tri, oh1, preferred_element_type=jnp.float32)
    csum2 = jnp.dot(tri, oh2, preferred_element_type=jnp.float32)
    total1 = jnp.sum(oh1, axis=0, keepdims=True)
    total2 = jnp.sum(oh2, axis=0, keepdims=True)
    carry = carry_ref[...]
    rank1 = jnp.sum(jnp.where(oh1 > 0, csum1 + carry - 1.0, 0.0),
                    axis=1, keepdims=True)
    rank2 = jnp.sum(jnp.where(oh2 > 0, csum2 + carry + total1 - 1.0, 0.0),
                    axis=1, keepdims=True)
    r1o_ref[...] = rank1.astype(jnp.int32)
    r2o_ref[...] = rank2.astype(jnp.int32)
    new_carry = carry + total1 + total2
    carry_ref[...] = new_carry

    tilesf = jnp.floor((new_carry + (M - 1.0)) * (1.0 / M))
    r16 = jax.lax.broadcasted_iota(jnp.int32, (16, 16), 0)
    c16 = jax.lax.broadcasted_iota(jnp.int32, (16, 16), 1)
    triu = (r16 <= c16).astype(jnp.float32)
    incl = jnp.dot(tilesf, triu, preferred_element_type=jnp.float32)
    starts = incl - tilesf
    cnt_ref[...] = starts.astype(jnp.int32)

    lane16 = jax.lax.broadcasted_iota(jnp.int32, (1, 16), 1)
    tt = jax.lax.broadcasted_iota(jnp.int32, (32, 16), 0).astype(jnp.float32)
    cmp = jnp.where((starts <= tt) & (lane16 < E), 1.0, 0.0)
    te = jnp.sum(cmp, axis=1, keepdims=True) - 1.0
    n_tiles = jnp.sum(jnp.where(lane16 == E, starts, 0.0), axis=1,
                      keepdims=True)
    row32 = jax.lax.broadcasted_iota(jnp.int32, (32, 1), 0)
    te_ref[...] = jnp.where(row32 == 31, n_tiles, te).astype(jnp.int32)


def _run_router(hidden, feats, ln_gamma, ln_beta, rW1, rb1, rW2, rb2):
    bm = 256
    grid = (B // bm,)
    out_shapes = (
        jax.ShapeDtypeStruct((B, D), jnp.float32),
        jax.ShapeDtypeStruct((B, 128), jnp.float32),
        jax.ShapeDtypeStruct((B, 128), jnp.float32),
        jax.ShapeDtypeStruct((B, 1), jnp.int32),
        jax.ShapeDtypeStruct((B, 1), jnp.int32),
        jax.ShapeDtypeStruct((B, 1), jnp.int32),
        jax.ShapeDtypeStruct((B, 1), jnp.int32),
        jax.ShapeDtypeStruct((1, 16), jnp.int32),
        jax.ShapeDtypeStruct((32, 1), jnp.int32),
    )

    def bspec():
        return pl.BlockSpec((bm, 1), lambda i: (i, 0))

    return pl.pallas_call(
        _router_body,
        grid=grid,
        in_specs=[
            pl.BlockSpec((bm, D), lambda i: (i, 0)),
            pl.BlockSpec((bm, FD), lambda i: (i, 0)),
            pl.BlockSpec((D,), lambda i: (0,)),
            pl.BlockSpec((D,), lambda i: (0,)),
            pl.BlockSpec((RIN, RH), lambda i: (0, 0)),
            pl.BlockSpec((RH,), lambda i: (0,)),
            pl.BlockSpec((RH, E), lambda i: (0, 0)),
            pl.BlockSpec((E,), lambda i: (0,)),
        ],
        out_specs=(
            pl.BlockSpec((bm, D), lambda i: (i, 0)),
            pl.BlockSpec((bm, 128), lambda i: (i, 0)),
            pl.BlockSpec((bm, 128), lambda i: (i, 0)),
            bspec(), bspec(), bspec(), bspec(),
            pl.BlockSpec((1, 16), lambda i: (0, 0)),
            pl.BlockSpec((32, 1), lambda i: (0, 0)),
        ),
        out_shape=out_shapes,
        scratch_shapes=[pltpu.VMEM((1, 16), jnp.float32)],
        interpret=INTERPRET,
    )(hidden, feats, ln_gamma, ln_beta, rW1, rb1, rW2, rb2)


def _dispatch_body(cnt_hbm, i1_hbm, i2_hbm, r1_hbm, r2_hbm, g1_hbm, g2_hbm,
                   h_hbm, f2_hbm,
                   xh_hbm, xf_hbm, s1_hbm, s2_hbm,
                   rsg_v, ids_v, rnk_v, slotb_v, rows_v, fr_v,
                   sem_h, sem_f, sem_fs, sem_x, sem_g):
    wid = lax.axis_index("s") * 2 + lax.axis_index("c")
    base = wid * TPW
    for j in range(TPW // CH):
        off = base + j * CH
        ch = pltpu.async_copy(h_hbm.at[pl.ds(off, CH)], rows_v, sem_h)
        for k in range(2):
            ik_hbm = i1_hbm if k == 0 else i2_hbm
            rk_hbm = r1_hbm if k == 0 else r2_hbm
            gk_hbm = g1_hbm if k == 0 else g2_hbm
            sk_hbm = s1_hbm if k == 0 else s2_hbm
            pltpu.sync_copy(ik_hbm.at[pl.ds(off, CH)], ids_v)
            pltpu.sync_copy(rk_hbm.at[pl.ds(off, CH)], rnk_v)
            cgl = pltpu.async_copy(gk_hbm.at[pl.ds(off, CH)],
                                   fr_v.at[:, pl.ds(128, 128)], sem_g)
            ids = ids_v[...]
            pltpu.sync_copy(cnt_hbm.at[ids], rsg_v)
            slot = (rsg_v[...] << 8) + rnk_v[...]
            slotb_v[...] = slot
            pltpu.sync_copy(slotb_v, sk_hbm.at[pl.ds(off, CH)])
            toks = jax.lax.broadcasted_iota(jnp.int32, (16,), 0) + off
            fidx = toks * 2 + (ids >> 2)
            cf = pltpu.async_copy(f2_hbm.at[fidx],
                                  fr_v.at[:, pl.ds(0, 128)], sem_f)
            cgl.wait()
            cf.wait()
            cf2 = pltpu.async_copy(fr_v, xf_hbm.at[slot], sem_fs)
            if k == 0:
                ch.wait()
            cx = pltpu.async_copy(rows_v, xh_hbm.at[slot], sem_x)
            cf2.wait()
            cx.wait()


def _run_dispatch(cnt16, i1, i2, r1, r2, g1b, g2b, h, f2):
    mesh = plsc.VectorSubcoreMesh(core_axis_name="c", subcore_axis_name="s")
    out_type = (
        jax.ShapeDtypeStruct((NP, D), jnp.float32),
        jax.ShapeDtypeStruct((NP, 256), jnp.float32),
        jax.ShapeDtypeStruct((B,), jnp.int32),
        jax.ShapeDtypeStruct((B,), jnp.int32),
    )
    f = pl.kernel(
        _dispatch_body,
        mesh=mesh,
        out_type=out_type,
        scratch_types=[
            pltpu.VMEM((16,), jnp.int32),
            pltpu.VMEM((16,), jnp.int32),
            pltpu.VMEM((16,), jnp.int32),
            pltpu.VMEM((16,), jnp.int32),
            pltpu.VMEM((CH, D), jnp.float32),
            pltpu.VMEM((CH, 256), jnp.float32),
            pltpu.SemaphoreType.DMA,
            pltpu.SemaphoreType.DMA,
            pltpu.SemaphoreType.DMA,
            pltpu.SemaphoreType.DMA,
            pltpu.SemaphoreType.DMA,
        ],
    )
    return f(cnt16, i1, i2, r1, r2, g1b, g2b, h, f2)


def _bdot(a, b):
    return jnp.dot(a.astype(jnp.bfloat16), b.astype(jnp.bfloat16),
                   preferred_element_type=jnp.float32)


def _expert_body(te_ref, xh_ref, xf_ref, w1h_ref, w1f_ref, b1_ref,
                 w2_ref, b2_ref, w3_ref, b3_ref, y_ref):
    @pl.when(pl.program_id(0) < te_ref[31])
    def _():
        x1 = _bdot(xh_ref[...], w1h_ref[0])
        x1 = x1 + _bdot(xf_ref[:, :128], w1f_ref[0])
        h1 = _gelu(x1 + b1_ref[0])
        h2 = _gelu(_bdot(h1, w2_ref[0]) + b2_ref[0])
        y = _bdot(h2, w3_ref[0]) + b3_ref[0]
        y_ref[...] = y * xf_ref[:, 128:129]


def _run_experts(te, xh, xf, We1, W1fp, be1, We2, be2, We3, be3):
    grid_spec = pltpu.PrefetchScalarGridSpec(
        num_scalar_prefetch=1,
        grid=(NT,),
        in_specs=[
            pl.BlockSpec((M, D), lambda i, te: (i, 0)),
            pl.BlockSpec((M, 256), lambda i, te: (i, 0)),
            pl.BlockSpec((1, D, H), lambda i, te: (te[i], 0, 0)),
            pl.BlockSpec((1, 128, H), lambda i, te: (te[i], 0, 0)),
            pl.BlockSpec((1, 1, H), lambda i, te: (te[i], 0, 0)),
            pl.BlockSpec((1, H, H), lambda i, te: (te[i], 0, 0)),
            pl.BlockSpec((1, 1, H), lambda i, te: (te[i], 0, 0)),
            pl.BlockSpec((1, H, D), lambda i, te: (te[i], 0, 0)),
            pl.BlockSpec((1, 1, D), lambda i, te: (te[i], 0, 0)),
        ],
        out_specs=pl.BlockSpec((M, D), lambda i, te: (i, 0)),
    )
    return pl.pallas_call(
        _expert_body,
        grid_spec=grid_spec,
        out_shape=jax.ShapeDtypeStruct((NP, D), jnp.float32),
        interpret=INTERPRET,
    )(te, xh, xf, We1, W1fp, be1, We2, be2, We3, be3)


def _combine_body(hid_hbm, yg_hbm, s1_hbm, s2_hbm, al_hbm,
                  out_hbm, s1_v, s2_v, y1_v, y2_v, hid_v, al_v, sem, sem2,
                  sem3, semw):
    wid = lax.axis_index("s") * 2 + lax.axis_index("c")
    base = wid * TPW
    pltpu.sync_copy(al_hbm, al_v)
    av = al_v[...]
    pltpu.sync_copy(s1_hbm.at[pl.ds(base, TPW)], s1_v)
    pltpu.sync_copy(s2_hbm.at[pl.ds(base, TPW)], s2_v)
    cw = None
    for j in range(TPW // CH):
        off = base + j * CH
        sl = pl.ds(j * CH, CH)
        c1 = pltpu.async_copy(yg_hbm.at[s1_v[sl]], y1_v, sem)
        c2 = pltpu.async_copy(yg_hbm.at[s2_v[sl]], y2_v, sem2)
        if cw is not None:
            cw.wait()
        c3 = pltpu.async_copy(hid_hbm.at[pl.ds(off, CH)], hid_v, sem3)
        c1.wait()
        c2.wait()
        c3.wait()
        for r in range(CH):
            def body(c, _):
                for u in range(8):
                    s = pl.ds(c * 128 + u * 16, 16)
                    hid_v[r, s] = hid_v[r, s] + av * (y1_v[r, s] + y2_v[r, s])
                return 0
            lax.fori_loop(0, D // 128, body, 0)
        cw = pltpu.async_copy(hid_v, out_hbm.at[pl.ds(off, CH)], semw)
    cw.wait()


def _run_combine(hidden, yg, s1, s2, alpha16):
    mesh = plsc.VectorSubcoreMesh(core_axis_name="c", subcore_axis_name="s")
    f = pl.kernel(
        _combine_body,
        mesh=mesh,
        out_type=jax.ShapeDtypeStruct((B, D), jnp.float32),
        scratch_types=[
            pltpu.VMEM((TPW,), jnp.int32),
            pltpu.VMEM((TPW,), jnp.int32),
            pltpu.VMEM((CH, D), jnp.float32),
            pltpu.VMEM((CH, D), jnp.float32),
            pltpu.VMEM((CH, D), jnp.float32),
            pltpu.VMEM((16,), jnp.float32),
            pltpu.SemaphoreType.DMA,
            pltpu.SemaphoreType.DMA,
            pltpu.SemaphoreType.DMA,
            pltpu.SemaphoreType.DMA,
        ],
    )
    return f(hidden, yg, s1, s2, alpha16)


def kernel(hidden, feature_bank, expert_bank_idx, ln_gamma, ln_beta,
           rW1, rb1, rW2, rb2, We1, be1, We2, be2, We3, be3, alpha):
    feats = feature_bank.reshape(B, FD)
    h, g1, g2, i1, i2, r1, r2, cnt, te = _run_router(
        hidden, feats, ln_gamma, ln_beta, rW1, rb1, rW2, rb2)

    f2 = feats.reshape(B * 2, 128)
    xh, xf, s1, s2 = _run_dispatch(
        cnt.reshape(16), i1.reshape(B), i2.reshape(B), r1.reshape(B),
        r2.reshape(B), g1, g2, h, f2)

    W1fp = jnp.matmul(_FSEL, We1[:, D:, :])
    yg = _run_experts(te.reshape(32), xh, xf, We1, W1fp,
                      be1.reshape(E, 1, H), We2, be2.reshape(E, 1, H),
                      We3, be3.reshape(E, 1, D))

    alpha16 = jnp.full((16,), 1.0, jnp.float32) * alpha
    return _run_combine(hidden, yg, s1, s2, alpha16)
